# baseline (speedup 1.0000x reference)
import functools

import jax
import jax.numpy as jnp
from jax import lax
from jax.experimental import pallas as pl
from jax.experimental.pallas import tpu as pltpu
from jax.experimental.pallas import tpu_sc as plsc

B, S, D = 4, 4096, 768
N = B * S
NC, NS = 2, 16
NW = NC * NS
TPW = N // NW
T = 32
NB = TPW // T
LANES = 16
DV = D // LANES


def _sc_body(x_hbm, pos_hbm, vidx_hbm, lidx_hbm, var_hbm, time_hbm,
             out_hbm, vidx_v, lidx_v, xb, pb, vb, tb,
             sem_x, sem_p, sem_v, sem_t):
  wid = lax.axis_index("s") * NC + lax.axis_index("c")
  base = pl.multiple_of(wid * TPW, TPW)

  def run_block(blk, _):
    tok = pl.multiple_of(base + blk * T, T)
    tok_slice = pl.ds(tok, T)
    pltpu.sync_copy(vidx_hbm.at[tok_slice], vidx_v)
    pltpu.sync_copy(lidx_hbm.at[tok_slice], lidx_v)
    cp_x = pltpu.async_copy(x_hbm.at[tok_slice], xb, sem_x)
    cp_p = pltpu.async_copy(pos_hbm.at[tok_slice], pb, sem_p)
    cp_v = pltpu.async_copy(var_hbm.at[vidx_v], vb, sem_v)
    cp_t = pltpu.async_copy(time_hbm.at[lidx_v], tb, sem_t)
    cp_x.wait()
    cp_p.wait()
    cp_v.wait()
    cp_t.wait()

    def add_step(i, _):
      t = i // DV
      d = (i % DV) * LANES
      s = pl.ds(d, LANES)
      xb[t, s] = xb[t, s] + pb[t, s] + vb[t, s] + tb[t, s]
      return 0

    lax.fori_loop(0, T * DV, add_step, 0)
    pltpu.sync_copy(xb, out_hbm.at[tok_slice])
    return 0

  lax.fori_loop(0, NB, run_block, 0)


@jax.jit
def _sc_embed(x2, pos2, vidx, lidx, var_table, time_table):
  mesh = plsc.VectorSubcoreMesh(
      core_axis_name="c", subcore_axis_name="s",
      num_cores=NC, num_subcores=NS)
  return pl.kernel(
      _sc_body,
      out_type=jax.ShapeDtypeStruct((N, D), jnp.float32),
      mesh=mesh,
      scratch_types=[
          pltpu.VMEM((T,), jnp.int32),
          pltpu.VMEM((T,), jnp.int32),
          pltpu.VMEM((T, D), jnp.float32),
          pltpu.VMEM((T, D), jnp.float32),
          pltpu.VMEM((T, D), jnp.float32),
          pltpu.VMEM((T, D), jnp.float32),
          pltpu.SemaphoreType.DMA,
          pltpu.SemaphoreType.DMA,
          pltpu.SemaphoreType.DMA,
          pltpu.SemaphoreType.DMA,
      ],
  )(x2, pos2, vidx, lidx, var_table, time_table)


def kernel(x, variable_seq, pos_emb, lead_time_seq, var_table, time_table):
  x2 = x.reshape(N, D)
  pos2 = pos_emb.reshape(N, D)
  vidx = variable_seq.reshape(N).astype(jnp.int32)
  lidx = lead_time_seq.reshape(N).astype(jnp.int32)
  out = _sc_embed(x2, pos2, vidx, lidx, var_table, time_table)
  return out.reshape(B, S, D)


# speedup vs baseline: 2.1070x; 2.1070x over previous
import jax
import jax.numpy as jnp
from jax import lax
from jax.experimental import pallas as pl
from jax.experimental.pallas import tpu as pltpu
from jax.experimental.pallas import tpu_sc as plsc

B, S, D = 4, 4096, 768
N = B * S
NC, NS = 2, 16
NW = NC * NS
TPW = N // NW
T = 16
NB = TPW // T
LANES = 16
DV = D // LANES


def _sc_body(x_hbm, pos_hbm, vidx_hbm, lidx_hbm, var_hbm, time_hbm,
             out_hbm, vidx_all, lidx_all,
             xb0, pb0, vb0, tb0, xb1, pb1, vb1, tb1,
             sem_in0, sem_in1, sem_out0, sem_out1):
  wid = lax.axis_index("s") * NC + lax.axis_index("c")
  base = pl.multiple_of(wid * TPW, TPW)

  pltpu.sync_copy(vidx_hbm.at[pl.ds(base, TPW)], vidx_all)
  pltpu.sync_copy(lidx_hbm.at[pl.ds(base, TPW)], lidx_all)

  bufs = ((xb0, pb0, vb0, tb0, sem_in0, sem_out0),
          (xb1, pb1, vb1, tb1, sem_in1, sem_out1))

  def tok_slice(blk):
    return pl.ds(pl.multiple_of(base + blk * T, T), T)

  def fire(blk):
    xb, pb, vb, tb, sem_in, _ = bufs[blk % 2]
    loc = pl.ds(blk * T, T)
    return (
        pltpu.async_copy(x_hbm.at[tok_slice(blk)], xb, sem_in),
        pltpu.async_copy(pos_hbm.at[tok_slice(blk)], pb, sem_in),
        pltpu.async_copy(var_hbm.at[vidx_all.at[loc]], vb, sem_in),
        pltpu.async_copy(time_hbm.at[lidx_all.at[loc]], tb, sem_in),
    )

  pending = fire(0)
  out_cp = {}
  for blk in range(NB):
    xb, pb, vb, tb, _, sem_out = bufs[blk % 2]
    if blk + 1 < NB:
      if blk - 1 >= 0:
        out_cp.pop(blk - 1).wait()
      nxt = fire(blk + 1)
    for cp in pending:
      cp.wait()
    if blk + 1 < NB:
      pending = nxt

    @plsc.parallel_loop(0, T * DV, unroll=4)
    def add_step(i):
      t = i & (T - 1)
      d = (i >> 4) * LANES
      s = pl.ds(d, LANES)
      xb[t, s] = xb[t, s] + pb[t, s] + vb[t, s] + tb[t, s]

    out_cp[blk] = pltpu.async_copy(xb, out_hbm.at[tok_slice(blk)], sem_out)
  out_cp.pop(NB - 1).wait()


@jax.jit
def _sc_embed(x2, pos2, vidx, lidx, var_table, time_table):
  mesh = plsc.VectorSubcoreMesh(
      core_axis_name="c", subcore_axis_name="s",
      num_cores=NC, num_subcores=NS)
  return pl.kernel(
      _sc_body,
      out_type=jax.ShapeDtypeStruct((N, D), jnp.float32),
      mesh=mesh,
      scratch_types=[
          pltpu.VMEM((TPW,), jnp.int32),
          pltpu.VMEM((TPW,), jnp.int32),
          pltpu.VMEM((T, D), jnp.float32),
          pltpu.VMEM((T, D), jnp.float32),
          pltpu.VMEM((T, D), jnp.float32),
          pltpu.VMEM((T, D), jnp.float32),
          pltpu.VMEM((T, D), jnp.float32),
          pltpu.VMEM((T, D), jnp.float32),
          pltpu.VMEM((T, D), jnp.float32),
          pltpu.VMEM((T, D), jnp.float32),
          pltpu.SemaphoreType.DMA,
          pltpu.SemaphoreType.DMA,
          pltpu.SemaphoreType.DMA,
          pltpu.SemaphoreType.DMA,
      ],
  )(x2, pos2, vidx, lidx, var_table, time_table)


def kernel(x, variable_seq, pos_emb, lead_time_seq, var_table, time_table):
  x2 = x.reshape(N, D)
  pos2 = pos_emb.reshape(N, D)
  vidx = variable_seq.reshape(N).astype(jnp.int32)
  lidx = lead_time_seq.reshape(N).astype(jnp.int32)
  out = _sc_embed(x2, pos2, vidx, lidx, var_table, time_table)
  return out.reshape(B, S, D)


# speedup vs baseline: 2.7610x; 1.3104x over previous
import jax
import jax.numpy as jnp
from jax import lax
from jax.experimental import pallas as pl
from jax.experimental.pallas import tpu as pltpu
from jax.experimental.pallas import tpu_sc as plsc

B, S, D = 4, 4096, 768
N = B * S
NC, NS = 2, 16
NW = NC * NS
NH = 2
DH = D // NH
NG = NW // NH
TPG = N // NG
T = 16
NBT = TPG // T
LANES = 16
DV = DH // LANES


def _sc_body(x_hbm, pos_hbm, vidx_hbm, lidx_hbm, var_hbm, time_hbm,
             out_hbm, vidx_all, lidx_all,
             xb0, pb0, ob0, xb1, pb1, ob1, var_t, time_t,
             sem_in0, sem_in1, sem_out0, sem_out1):
  wid = lax.axis_index("s") * NC + lax.axis_index("c")
  g = wid // NH
  h = wid % NH
  tok0 = pl.multiple_of(g * TPG, TPG)
  col0 = pl.multiple_of(h * DH, DH)
  cols = pl.ds(col0, DH)

  pltpu.sync_copy(var_hbm.at[:, cols], var_t)
  pltpu.sync_copy(time_hbm.at[:, cols], time_t)
  pltpu.sync_copy(vidx_hbm.at[pl.ds(tok0, TPG)], vidx_all)
  pltpu.sync_copy(lidx_hbm.at[pl.ds(tok0, TPG)], lidx_all)

  bufs = ((xb0, pb0, ob0, sem_in0, sem_out0),
          (xb1, pb1, ob1, sem_in1, sem_out1))

  def rows(blk):
    return pl.ds(tok0 + blk * T, T)

  def fire_in(blk, xb, pb, sem):
    pltpu.async_copy(x_hbm.at[rows(blk), cols], xb, sem)
    pltpu.async_copy(pos_hbm.at[rows(blk), cols], pb, sem)

  dios = [jnp.arange(dv * LANES, (dv + 1) * LANES, dtype=jnp.int32)
          for dv in range(DV)]

  dnums = lax.GatherDimensionNumbers(
      offset_dims=(), collapsed_slice_dims=(0,), start_index_map=(0,))

  def lane_bcast(vec, lane):
    return lax.gather(vec, lane[:, None], dnums, (1,),
                      mode=lax.GatherScatterMode.PROMISE_IN_BOUNDS)

  def half(parity, blk):
    xb, pb, ob, sem_in, sem_out = bufs[parity]
    pltpu.make_async_copy(x_hbm.at[rows(blk), cols], xb, sem_in).wait()
    pltpu.make_async_copy(pos_hbm.at[rows(blk), cols], pb, sem_in).wait()

    @pl.when(blk >= 2)
    def _():
      pltpu.make_async_copy(ob, out_hbm.at[rows(blk), cols], sem_out).wait()

    seg = pl.ds(pl.multiple_of(blk * T, T), LANES)
    vsegv = vidx_all[seg]
    lsegv = lidx_all[seg]

    @plsc.parallel_loop(0, T)
    def token_step(t):
      lane = jnp.broadcast_to(t, (LANES,))
      rv = lane_bcast(vsegv, lane)
      rt = lane_bcast(lsegv, lane)
      for dv in range(DV):
        s = pl.ds(dv * LANES, LANES)
        varv = plsc.load_gather(var_t, [rv, dios[dv]])
        timv = plsc.load_gather(time_t, [rt, dios[dv]])
        ob[t, s] = xb[t, s] + pb[t, s] + varv + timv

    pltpu.async_copy(ob, out_hbm.at[rows(blk), cols], sem_out)

    @pl.when(blk + 2 < NBT)
    def _():
      fire_in(blk + 2, xb, pb, sem_in)

  fire_in(0, xb0, pb0, sem_in0)
  fire_in(1, xb1, pb1, sem_in1)

  def pair(gg, _):
    half(0, gg * 2)
    half(1, gg * 2 + 1)
    return 0

  lax.fori_loop(0, NBT // 2, pair, 0)
  pltpu.make_async_copy(ob0, out_hbm.at[rows(0), cols], sem_out0).wait()
  pltpu.make_async_copy(ob1, out_hbm.at[rows(1), cols], sem_out1).wait()


@jax.jit
def _sc_embed(x2, pos2, vidx, lidx, var_table, time_table):
  mesh = plsc.VectorSubcoreMesh(
      core_axis_name="c", subcore_axis_name="s",
      num_cores=NC, num_subcores=NS)
  return pl.kernel(
      _sc_body,
      out_type=jax.ShapeDtypeStruct((N, D), jnp.float32),
      mesh=mesh,
      compiler_params=pltpu.CompilerParams(needs_layout_passes=False),
      scratch_types=[
          pltpu.VMEM((TPG,), jnp.int32),
          pltpu.VMEM((TPG,), jnp.int32),
          pltpu.VMEM((T, DH), jnp.float32),
          pltpu.VMEM((T, DH), jnp.float32),
          pltpu.VMEM((T, DH), jnp.float32),
          pltpu.VMEM((T, DH), jnp.float32),
          pltpu.VMEM((T, DH), jnp.float32),
          pltpu.VMEM((T, DH), jnp.float32),
          pltpu.VMEM((100, DH), jnp.float32),
          pltpu.VMEM((100, DH), jnp.float32),
          pltpu.SemaphoreType.DMA,
          pltpu.SemaphoreType.DMA,
          pltpu.SemaphoreType.DMA,
          pltpu.SemaphoreType.DMA,
      ],
  )(x2, pos2, vidx, lidx, var_table, time_table)


def kernel(x, variable_seq, pos_emb, lead_time_seq, var_table, time_table):
  x2 = x.reshape(N, D)
  pos2 = pos_emb.reshape(N, D)
  vidx = variable_seq.reshape(N).astype(jnp.int32)
  lidx = lead_time_seq.reshape(N).astype(jnp.int32)
  out = _sc_embed(x2, pos2, vidx, lidx, var_table, time_table)
  return out.reshape(B, S, D)
